# baseline (speedup 1.0000x reference)
import functools

import numpy as np
import jax
import jax.numpy as jnp
from jax.experimental import pallas as pl
from jax.experimental.pallas import tpu as pltpu

_VMEM_LIMIT = 100 * 1024 * 1024


def _prep_weight(weight, gain=1.0, eps=1e-4):
    w = weight.astype(jnp.float32)
    reduce_dims = tuple(range(1, w.ndim))
    fan_in = int(np.prod(w.shape[1:]))
    norm = jnp.sqrt(jnp.sum(w * w, axis=reduce_dims, keepdims=True))
    norm = eps + norm * np.sqrt(1.0 / fan_in)
    return w / norm * (gain / np.sqrt(fan_in))


def _conv_kernel(x_ref, w_ref, o_ref, *scratch, H, W, pad, B):
    S = H * W
    cin = x_ref.shape[1]
    cout = o_ref.shape[1]
    col = jax.lax.broadcasted_iota(jnp.int32, (1, S), 1) % W

    def build(b):
        xpad_ref = scratch[2 * (b % 2)]
        xs_ref = scratch[2 * (b % 2) + 1]
        xb = x_ref[b].reshape(cin, S).astype(jnp.bfloat16)
        xpad_ref[:, :pad] = jnp.zeros((cin, pad), jnp.bfloat16)
        xpad_ref[:, pad:pad + S] = xb
        xpad_ref[:, pad + S:] = jnp.zeros((cin, pad), jnp.bfloat16)
        for t in range(9):
            kh, kw = t // 3, t % 3
            off = (kh - 1) * W + (kw - 1)
            xs = xpad_ref[:, pl.ds(pad + off, S)]
            if kw == 0:
                xs = jnp.where(col == 0, jnp.bfloat16(0), xs)
            elif kw == 2:
                xs = jnp.where(col == W - 1, jnp.bfloat16(0), xs)
            xs_ref[t * cin:(t + 1) * cin, :] = xs

    def matmul(b):
        xs_ref = scratch[2 * (b % 2) + 1]
        acc = jax.lax.dot_general(
            w_ref[...], xs_ref[...],
            dimension_numbers=(((1,), (0,)), ((), ())),
            preferred_element_type=jnp.float32)
        o_ref[b] = acc.reshape(cout, S // 128, 128)

    build(0)
    for b in range(1, B):
        build(b)
        matmul(b - 1)
    matmul(B - 1)


def kernel(x, weight):
    N, Cin, H, W = x.shape
    Cout = weight.shape[0]
    S = H * W
    pad = 64
    B = 4
    assert weight.shape[2] == 3 and weight.shape[3] == 3 and N % B == 0
    assert S % 128 == 0

    w = _prep_weight(weight, gain=1.0)
    w2 = jnp.transpose(w, (0, 2, 3, 1)).reshape(Cout, 9 * Cin).astype(jnp.bfloat16)
    x5 = x.reshape(N, Cin, S // 128, 128)

    scratch = []
    for _ in range(2):
        scratch.append(pltpu.VMEM((Cin, pad + S + pad), jnp.bfloat16))
        scratch.append(pltpu.VMEM((9 * Cin, S), jnp.bfloat16))

    body = functools.partial(_conv_kernel, H=H, W=W, pad=pad, B=B)
    out = pl.pallas_call(
        body,
        out_shape=jax.ShapeDtypeStruct((N, Cout, S // 128, 128), x.dtype),
        grid_spec=pltpu.PrefetchScalarGridSpec(
            num_scalar_prefetch=0,
            grid=(N // B,),
            in_specs=[
                pl.BlockSpec((B, Cin, S // 128, 128), lambda n: (n, 0, 0, 0)),
                pl.BlockSpec((Cout, 9 * Cin), lambda n: (0, 0)),
            ],
            out_specs=pl.BlockSpec((B, Cout, S // 128, 128), lambda n: (n, 0, 0, 0)),
            scratch_shapes=scratch),
        compiler_params=pltpu.CompilerParams(
            dimension_semantics=("parallel",),
            vmem_limit_bytes=_VMEM_LIMIT),
    )(x5, w2)
    return out.reshape(N, Cout, H, W)


# speedup vs baseline: 1.0240x; 1.0240x over previous
import functools

import numpy as np
import jax
import jax.numpy as jnp
from jax.experimental import pallas as pl
from jax.experimental.pallas import tpu as pltpu

_VMEM_LIMIT = 100 * 1024 * 1024


def _prep_weight(weight, gain=1.0, eps=1e-4):
    w = weight.astype(jnp.float32)
    reduce_dims = tuple(range(1, w.ndim))
    fan_in = int(np.prod(w.shape[1:]))
    norm = jnp.sqrt(jnp.sum(w * w, axis=reduce_dims, keepdims=True))
    norm = eps + norm * np.sqrt(1.0 / fan_in)
    return w / norm * (gain / np.sqrt(fan_in))


def _conv_kernel(x_ref, w_ref, o_ref, xpad_ref, xs_ref, *, H, W, pad, B):
    S = H * W
    cin = x_ref.shape[1]
    cout = o_ref.shape[1]
    col = jax.lax.broadcasted_iota(jnp.int32, (1, S), 1) % W

    for b in range(B):
        xb = x_ref[b].reshape(cin, S).astype(jnp.bfloat16)
        xpad_ref[b, :, :pad] = jnp.zeros((cin, pad), jnp.bfloat16)
        xpad_ref[b, :, pad:pad + S] = xb
        xpad_ref[b, :, pad + S:] = jnp.zeros((cin, pad), jnp.bfloat16)
        for t in range(9):
            kh, kw = t // 3, t % 3
            off = (kh - 1) * W + (kw - 1)
            xs = xpad_ref[b, :, pl.ds(pad + off, S)]
            if kw == 0:
                xs = jnp.where(col == 0, jnp.bfloat16(0), xs)
            elif kw == 2:
                xs = jnp.where(col == W - 1, jnp.bfloat16(0), xs)
            xs_ref[t * cin:(t + 1) * cin, b * S:(b + 1) * S] = xs
        acc = jax.lax.dot_general(
            w_ref[...], xs_ref[:, b * S:(b + 1) * S],
            dimension_numbers=(((1,), (0,)), ((), ())),
            preferred_element_type=jnp.float32)
        o_ref[b] = acc.reshape(cout, S // 128, 128)


def kernel(x, weight):
    N, Cin, H, W = x.shape
    Cout = weight.shape[0]
    S = H * W
    pad = 64
    B = 8
    assert weight.shape[2] == 3 and weight.shape[3] == 3 and N % B == 0
    assert S % 128 == 0

    w = _prep_weight(weight, gain=1.0)
    w2 = jnp.transpose(w, (0, 2, 3, 1)).reshape(Cout, 9 * Cin).astype(jnp.bfloat16)
    x5 = x.reshape(N, Cin, S // 128, 128)

    body = functools.partial(_conv_kernel, H=H, W=W, pad=pad, B=B)
    out = pl.pallas_call(
        body,
        out_shape=jax.ShapeDtypeStruct((N, Cout, S // 128, 128), x.dtype),
        grid_spec=pltpu.PrefetchScalarGridSpec(
            num_scalar_prefetch=0,
            grid=(N // B,),
            in_specs=[
                pl.BlockSpec((B, Cin, S // 128, 128), lambda n: (n, 0, 0, 0)),
                pl.BlockSpec((Cout, 9 * Cin), lambda n: (0, 0)),
            ],
            out_specs=pl.BlockSpec((B, Cout, S // 128, 128), lambda n: (n, 0, 0, 0)),
            scratch_shapes=[
                pltpu.VMEM((B, Cin, pad + S + pad), jnp.bfloat16),
                pltpu.VMEM((9 * Cin, B * S), jnp.bfloat16),
            ]),
        compiler_params=pltpu.CompilerParams(
            dimension_semantics=("parallel",),
            vmem_limit_bytes=_VMEM_LIMIT),
    )(x5, w2)
    return out.reshape(N, Cout, H, W)


# speedup vs baseline: 1.0721x; 1.0470x over previous
import functools

import numpy as np
import jax
import jax.numpy as jnp
from jax.experimental import pallas as pl
from jax.experimental.pallas import tpu as pltpu

_VMEM_LIMIT = 100 * 1024 * 1024


def _prep_weight(weight, gain=1.0, eps=1e-4):
    w = weight.astype(jnp.float32)
    reduce_dims = tuple(range(1, w.ndim))
    fan_in = int(np.prod(w.shape[1:]))
    norm = jnp.sqrt(jnp.sum(w * w, axis=reduce_dims, keepdims=True))
    norm = eps + norm * np.sqrt(1.0 / fan_in)
    return w / norm * (gain / np.sqrt(fan_in))


def _conv_kernel(x_ref, w_ref, o_ref, xpad_ref, xs_ref, *, H, W, pad, B):
    S = H * W
    cin = x_ref.shape[1]
    cout = o_ref.shape[1]
    col = jax.lax.broadcasted_iota(jnp.int32, (1, S), 1) % W

    for b in range(B):
        xb = x_ref[b].reshape(cin, S).astype(jnp.bfloat16)
        xpad_ref[b, :, :pad] = jnp.zeros((cin, pad), jnp.bfloat16)
        xpad_ref[b, :, pad:pad + S] = xb
        xpad_ref[b, :, pad + S:] = jnp.zeros((cin, pad), jnp.bfloat16)
        for t in range(9):
            kh, kw = t // 3, t % 3
            off = (kh - 1) * W + (kw - 1)
            xs = xpad_ref[b, :, pl.ds(pad + off, S)]
            if kw == 0:
                xs = jnp.where(col == 0, jnp.bfloat16(0), xs)
            elif kw == 2:
                xs = jnp.where(col == W - 1, jnp.bfloat16(0), xs)
            xs_ref[t * cin:(t + 1) * cin, b * S:(b + 1) * S] = xs
        acc = jax.lax.dot_general(
            w_ref[...], xs_ref[:, b * S:(b + 1) * S],
            dimension_numbers=(((1,), (0,)), ((), ())),
            preferred_element_type=jnp.float32)
        o_ref[b] = acc.astype(jnp.bfloat16).reshape(cout, S // 128, 128)


def kernel(x, weight):
    N, Cin, H, W = x.shape
    Cout = weight.shape[0]
    S = H * W
    pad = 64
    B = 8
    assert weight.shape[2] == 3 and weight.shape[3] == 3 and N % B == 0
    assert S % 128 == 0

    w = _prep_weight(weight, gain=1.0)
    w2 = jnp.transpose(w, (0, 2, 3, 1)).reshape(Cout, 9 * Cin).astype(jnp.bfloat16)
    x5 = x.reshape(N, Cin, S // 128, 128)

    body = functools.partial(_conv_kernel, H=H, W=W, pad=pad, B=B)
    out = pl.pallas_call(
        body,
        out_shape=jax.ShapeDtypeStruct((N, Cout, S // 128, 128), jnp.bfloat16),
        grid_spec=pltpu.PrefetchScalarGridSpec(
            num_scalar_prefetch=0,
            grid=(N // B,),
            in_specs=[
                pl.BlockSpec((B, Cin, S // 128, 128), lambda n: (n, 0, 0, 0)),
                pl.BlockSpec((Cout, 9 * Cin), lambda n: (0, 0)),
            ],
            out_specs=pl.BlockSpec((B, Cout, S // 128, 128), lambda n: (n, 0, 0, 0)),
            scratch_shapes=[
                pltpu.VMEM((B, Cin, pad + S + pad), jnp.bfloat16),
                pltpu.VMEM((9 * Cin, B * S), jnp.bfloat16),
            ]),
        compiler_params=pltpu.CompilerParams(
            dimension_semantics=("parallel",),
            vmem_limit_bytes=_VMEM_LIMIT),
    )(x5, w2)
    return out.reshape(N, Cout, H, W).astype(x.dtype)
